# baseline (speedup 1.0000x reference)
import functools

import jax
import jax.numpy as jnp
from jax import lax
from jax.experimental import pallas as pl
from jax.experimental.pallas import tpu as pltpu
from jax.experimental.pallas import tpu_sc as plsc

NUM_SEGMENTS = 16
D_MODEL = 1024

_INFO = plsc.get_sparse_core_info()
_NC, _NS = _INFO.num_cores, _INFO.num_subcores
_NW = _NC * _NS

_B = 4 * 4096
_BPW = _B // _NW
_C = 32
_NCHUNK = _BPW // _C
_NBUF = 3


@functools.partial(
    pl.kernel,
    mesh=plsc.VectorSubcoreMesh(core_axis_name="c", subcore_axis_name="s"),
    out_type=jax.ShapeDtypeStruct((_B, D_MODEL), jnp.float32),
    scratch_types=[
        pltpu.VMEM((_BPW,), jnp.int32),
        pltpu.VMEM((_NBUF, _C, D_MODEL), jnp.float32),
        pltpu.SemaphoreType.DMA((_NBUF,)),
        pltpu.SemaphoreType.DMA((_NBUF,)),
    ],
)
def _sc_lookup(seg_hbm, table_hbm, out_hbm, idx_v, bufs, gsem, wsem):
    wid = lax.axis_index("s") * _NC + lax.axis_index("c")
    base = wid * _BPW
    pltpu.sync_copy(seg_hbm.at[pl.ds(base, _BPW)], idx_v)

    def gather(chunk, b):
        return pltpu.async_copy(
            table_hbm.at[idx_v.at[pl.ds(chunk * _C, _C)]],
            bufs.at[b],
            gsem.at[b],
        )

    def write(chunk, b):
        return pltpu.async_copy(
            bufs.at[b],
            out_hbm.at[pl.ds(base + chunk * _C, _C)],
            wsem.at[b],
        )

    gh = [None] * _NBUF
    wh = [None] * _NBUF
    for b in range(_NBUF):
        gh[b] = gather(b, b)
    for c in range(_NCHUNK):
        b = c % _NBUF
        if c >= _NBUF:
            wh[b].wait()
            gh[b] = gather(c, b)
        gh[b].wait()
        wh[b] = write(c, b)
    for c in range(_NCHUNK - _NBUF, _NCHUNK):
        wh[c % _NBUF].wait()


def kernel(segment_ids, table):
    seg_flat = segment_ids.reshape(-1).astype(jnp.int32)
    out = _sc_lookup(seg_flat, table)
    return out.reshape(segment_ids.shape + (D_MODEL,))
